# baseline (speedup 1.0000x reference)
import functools

import jax
import jax.numpy as jnp
from jax import lax
from jax.experimental import pallas as pl
from jax.experimental.pallas import tpu as pltpu
from jax.experimental.pallas import tpu_sc as plsc

Q = 1024
K = 16384
BQ = 128
GRID = Q // BQ
NV = Q // 16


def _tc_body(qx, qy, qz, px, py, pz, nx, ny, nz, sc, ss_ref, sn_ref):
    qxv = qx[...]
    qyv = qy[...]
    qzv = qz[...]
    pxv = px[...]
    pyv = py[...]
    pzv = pz[...]
    nxv = nx[...]
    nyv = ny[...]
    nzv = nz[...]
    scv = sc[...]

    dx = qxv - pxv
    dy = qyv - pyv
    dz = qzv - pzv
    d2 = dx * dx + dy * dy + dz * dz
    d = jnp.sqrt(d2)

    m = jnp.full((BQ, 1), -jnp.inf, jnp.float32)
    for _ in range(8):
        masked = jnp.where(d > m, d, jnp.inf)
        m = jnp.min(masked, axis=1, keepdims=True)
    le = d <= m

    cnt = jnp.sum(jnp.where(le, 1.0, 0.0), axis=1, keepdims=True)
    sx = jnp.sum(jnp.where(le, nxv, 0.0), axis=1, keepdims=True)
    sy = jnp.sum(jnp.where(le, nyv, 0.0), axis=1, keepdims=True)
    sz = jnp.sum(jnp.where(le, nzv, 0.0), axis=1, keepdims=True)
    xn_x = sx / cnt
    xn_y = sy / cnt
    xn_z = sz / cnt

    na = jnp.sqrt(xn_x * xn_x + xn_y * xn_y + xn_z * xn_z)
    nb = jnp.sqrt(nxv * nxv + nyv * nyv + nzv * nzv)
    dot = xn_x * nxv + xn_y * nyv + xn_z * nzv
    thr = 0.75 * jnp.maximum(na * nb, 1e-6)
    gt = dot > thr

    sn_ref[...] = jnp.sum(jnp.where(gt, 1.0, 0.0), axis=1, keepdims=True)
    ss_ref[...] = jnp.sum(jnp.where(gt, scv * jnp.exp(-d), 0.0),
                          axis=1, keepdims=True)


def _tc_dense(qx, qy, qz, px, py, pz, nx, ny, nz, sc):
    qspec = pl.BlockSpec((BQ, 1), lambda i: (i, 0))
    kspec = pl.BlockSpec((1, K), lambda i: (0, 0))
    ospec = pl.BlockSpec((BQ, 1), lambda i: (i, 0))
    return pl.pallas_call(
        _tc_body,
        grid=(GRID,),
        in_specs=[qspec, qspec, qspec] + [kspec] * 7,
        out_specs=[ospec, ospec],
        out_shape=[
            jax.ShapeDtypeStruct((Q, 1), jnp.float32),
            jax.ShapeDtypeStruct((Q, 1), jnp.float32),
        ],
    )(qx, qy, qz, px, py, pz, nx, ny, nz, sc)


def _sc_compact_body(ss_hbm, sn_hbm, field_hbm, nzj_hbm,
                     ss_v, sn_v, nzj_v, field_v):
    c = lax.axis_index("c")
    s = lax.axis_index("s")

    @pl.when(jnp.logical_and(c == 0, s == 0))
    def _():
        pltpu.sync_copy(ss_hbm, ss_v)
        pltpu.sync_copy(sn_hbm, sn_v)

        zeros16 = jnp.zeros((16,), jnp.int32)
        for j in range(NV):
            nzj_v[pl.ds(j * 16, 16)] = zeros16

        base_iota = lax.iota(jnp.int32, 16)

        def step(j, off):
            v = sn_v[pl.ds(j * 16, 16)]
            msk = v != 0.0
            idxv = base_iota + j * 16
            plsc.store_compressed(nzj_v.at[pl.ds(off, 16)], idxv, mask=msk)
            pc = plsc.all_reduce_population_count(msk)
            return off + pc[0]

        lax.fori_loop(0, NV, step, jnp.int32(0))

        def emit(j, carry):
            idxv = nzj_v[pl.ds(j * 16, 16)]
            ssg = plsc.load_gather(ss_v, [idxv])
            sng = plsc.load_gather(sn_v, [idxv])
            field_v[pl.ds(j * 16, 16)] = ssg / sng
            return carry

        lax.fori_loop(0, NV, emit, jnp.int32(0))

        pltpu.sync_copy(field_v, field_hbm)
        pltpu.sync_copy(nzj_v, nzj_hbm)


@functools.partial(
    pl.kernel,
    mesh=plsc.VectorSubcoreMesh(core_axis_name="c", subcore_axis_name="s"),
    out_type=[
        jax.ShapeDtypeStruct((Q,), jnp.float32),
        jax.ShapeDtypeStruct((Q,), jnp.int32),
    ],
    scratch_types=[
        pltpu.VMEM((Q,), jnp.float32),
        pltpu.VMEM((Q,), jnp.float32),
        pltpu.VMEM((Q,), jnp.int32),
        pltpu.VMEM((Q,), jnp.float32),
    ],
    compiler_params=pltpu.CompilerParams(needs_layout_passes=False),
)
def _sc_compact(ss_hbm, sn_hbm, field_hbm, nzj_hbm,
                ss_v, sn_v, nzj_v, field_v):
    _sc_compact_body(ss_hbm, sn_hbm, field_hbm, nzj_hbm,
                     ss_v, sn_v, nzj_v, field_v)


def kernel(x_world, voxel_point, voxel_normal, score):
    q = x_world.reshape(Q, 3)
    p = voxel_point.reshape(K, 3)
    qx = q[:, 0].reshape(Q, 1)
    qy = q[:, 1].reshape(Q, 1)
    qz = q[:, 2].reshape(Q, 1)
    px = p[:, 0].reshape(1, K)
    py = p[:, 1].reshape(1, K)
    pz = p[:, 2].reshape(1, K)
    nx = voxel_normal[:, 0].reshape(1, K)
    ny = voxel_normal[:, 1].reshape(1, K)
    nz = voxel_normal[:, 2].reshape(1, K)
    sc = score.reshape(1, K)

    ss, sn = _tc_dense(qx, qy, qz, px, py, pz, nx, ny, nz, sc)
    field, nzj = _sc_compact(ss.reshape(Q), sn.reshape(Q))
    return field, nzj


# speedup vs baseline: 1.0120x; 1.0120x over previous
import functools

import jax
import jax.numpy as jnp
from jax import lax
from jax.experimental import pallas as pl
from jax.experimental.pallas import tpu as pltpu
from jax.experimental.pallas import tpu_sc as plsc

Q = 1024
K = 16384
BQ = 128
GRID = Q // BQ
NV = Q // 16


def _tc_body(qx, qy, qz, px, py, pz, nx, ny, nz, sc, n8, ss_ref, sn_ref):
    qxv = qx[...]
    qyv = qy[...]
    qzv = qz[...]
    pxv = px[...]
    pyv = py[...]
    pzv = pz[...]
    nxv = nx[...]
    nyv = ny[...]
    nzv = nz[...]
    scv = sc[...]

    dx = qxv - pxv
    dy = qyv - pyv
    dz = qzv - pzv
    d2 = dx * dx + dy * dy + dz * dz
    d = jnp.sqrt(d2)

    m = jnp.full((BQ, 1), -jnp.inf, jnp.float32)
    for _ in range(8):
        masked = jnp.where(d > m, d, jnp.inf)
        m = jnp.min(masked, axis=1, keepdims=True)
    le = d <= m

    lebf = jnp.where(le, 1.0, 0.0).astype(jnp.bfloat16)
    sums8 = jax.lax.dot_general(
        lebf, n8[...],
        (((1,), (0,)), ((), ())),
        preferred_element_type=jnp.float32,
    )
    cnt = sums8[:, 0:1]
    sx = sums8[:, 1:2] + sums8[:, 4:5]
    sy = sums8[:, 2:3] + sums8[:, 5:6]
    sz = sums8[:, 3:4] + sums8[:, 6:7]
    xn_x = sx / cnt
    xn_y = sy / cnt
    xn_z = sz / cnt

    na = jnp.sqrt(xn_x * xn_x + xn_y * xn_y + xn_z * xn_z)
    nb = jnp.sqrt(nxv * nxv + nyv * nyv + nzv * nzv)
    dot = xn_x * nxv + xn_y * nyv + xn_z * nzv
    thr = 0.75 * jnp.maximum(na * nb, 1e-6)
    gt = dot > thr

    sn_ref[...] = jnp.sum(jnp.where(gt, 1.0, 0.0), axis=1, keepdims=True)
    ss_ref[...] = jnp.sum(jnp.where(gt, scv * jnp.exp(-d), 0.0),
                          axis=1, keepdims=True)


def _tc_dense(qx, qy, qz, px, py, pz, nx, ny, nz, sc, n8):
    qspec = pl.BlockSpec((BQ, 1), lambda i: (i, 0))
    kspec = pl.BlockSpec((1, K), lambda i: (0, 0))
    n8spec = pl.BlockSpec((K, 8), lambda i: (0, 0))
    ospec = pl.BlockSpec((BQ, 1), lambda i: (i, 0))
    return pl.pallas_call(
        _tc_body,
        grid=(GRID,),
        in_specs=[qspec, qspec, qspec] + [kspec] * 7 + [n8spec],
        out_specs=[ospec, ospec],
        out_shape=[
            jax.ShapeDtypeStruct((Q, 1), jnp.float32),
            jax.ShapeDtypeStruct((Q, 1), jnp.float32),
        ],
    )(qx, qy, qz, px, py, pz, nx, ny, nz, sc, n8)


def _sc_compact_body(ss_hbm, sn_hbm, field_hbm, nzj_hbm,
                     ss_v, sn_v, nzj_v, field_v):
    c = lax.axis_index("c")
    s = lax.axis_index("s")

    @pl.when(jnp.logical_and(c == 0, s == 0))
    def _():
        pltpu.sync_copy(ss_hbm, ss_v)
        pltpu.sync_copy(sn_hbm, sn_v)

        zeros16 = jnp.zeros((16,), jnp.int32)
        for j in range(NV):
            nzj_v[pl.ds(j * 16, 16)] = zeros16

        base_iota = lax.iota(jnp.int32, 16)

        def step(j, off):
            v = sn_v[pl.ds(j * 16, 16)]
            msk = v != 0.0
            idxv = base_iota + j * 16
            plsc.store_compressed(nzj_v.at[pl.ds(off, 16)], idxv, mask=msk)
            pc = plsc.all_reduce_population_count(msk)
            return off + pc[0]

        lax.fori_loop(0, NV, step, jnp.int32(0))

        def emit(j, carry):
            idxv = nzj_v[pl.ds(j * 16, 16)]
            ssg = plsc.load_gather(ss_v, [idxv])
            sng = plsc.load_gather(sn_v, [idxv])
            field_v[pl.ds(j * 16, 16)] = ssg / sng
            return carry

        lax.fori_loop(0, NV, emit, jnp.int32(0))

        pltpu.sync_copy(field_v, field_hbm)
        pltpu.sync_copy(nzj_v, nzj_hbm)


@functools.partial(
    pl.kernel,
    mesh=plsc.VectorSubcoreMesh(core_axis_name="c", subcore_axis_name="s"),
    out_type=[
        jax.ShapeDtypeStruct((Q,), jnp.float32),
        jax.ShapeDtypeStruct((Q,), jnp.int32),
    ],
    scratch_types=[
        pltpu.VMEM((Q,), jnp.float32),
        pltpu.VMEM((Q,), jnp.float32),
        pltpu.VMEM((Q,), jnp.int32),
        pltpu.VMEM((Q,), jnp.float32),
    ],
    compiler_params=pltpu.CompilerParams(needs_layout_passes=False),
)
def _sc_compact(ss_hbm, sn_hbm, field_hbm, nzj_hbm,
                ss_v, sn_v, nzj_v, field_v):
    _sc_compact_body(ss_hbm, sn_hbm, field_hbm, nzj_hbm,
                     ss_v, sn_v, nzj_v, field_v)


def kernel(x_world, voxel_point, voxel_normal, score):
    q = x_world.reshape(Q, 3)
    p = voxel_point.reshape(K, 3)
    qx = q[:, 0].reshape(Q, 1)
    qy = q[:, 1].reshape(Q, 1)
    qz = q[:, 2].reshape(Q, 1)
    px = p[:, 0].reshape(1, K)
    py = p[:, 1].reshape(1, K)
    pz = p[:, 2].reshape(1, K)
    nx = voxel_normal[:, 0].reshape(1, K)
    ny = voxel_normal[:, 1].reshape(1, K)
    nz = voxel_normal[:, 2].reshape(1, K)
    sc = score.reshape(1, K)

    nh = voxel_normal.astype(jnp.bfloat16)
    nl = (voxel_normal - nh.astype(jnp.float32)).astype(jnp.bfloat16)
    n8 = jnp.concatenate(
        [jnp.ones((K, 1), jnp.bfloat16), nh, nl,
         jnp.zeros((K, 1), jnp.bfloat16)], axis=1)

    ss, sn = _tc_dense(qx, qy, qz, px, py, pz, nx, ny, nz, sc, n8)
    field, nzj = _sc_compact(ss.reshape(Q), sn.reshape(Q))
    return field, nzj
